# Initial kernel scaffold; baseline (speedup 1.0000x reference)
import jax
import jax.numpy as jnp
from jax.experimental import pallas as pl


def kernel(x, cluster, edge_index, identifier, valid_len, params):
    raise NotImplementedError("write your pallas kernel here")


# speedup vs baseline: 6.8769x; 6.8769x over previous
import jax
import jax.numpy as jnp
from jax.experimental import pallas as pl
from jax.experimental.pallas import tpu as pltpu

IN_CH = 8
HID = 64
SUB_W = 64
GG_W = 64
NUM_SUB_LAYERS = 3
BATCH = 32
TSL = 256
NUM_CLUSTERS = BATCH * TSL
NODES_PER = 32
N = NUM_CLUSTERS * NODES_PER
NPG = TSL * NODES_PER


def _ln(x, g, b):
    mu = jnp.mean(x, axis=-1, keepdims=True)
    xc = x - mu
    var = jnp.mean(xc * xc, axis=-1, keepdims=True)
    return xc * jax.lax.rsqrt(var + 1e-5) * g + b


def _mlp_block(h, W1, b1, g1, be1, W2, b2, g2, be2, Ws, bs, gs, bes):
    o = jnp.dot(h, W1, preferred_element_type=jnp.float32) + b1
    o = jax.nn.relu(_ln(o, g1, be1))
    o = jnp.dot(o, W2, preferred_element_type=jnp.float32) + b2
    o = _ln(o, g2, be2)
    sc = _ln(jnp.dot(h, Ws, preferred_element_type=jnp.float32) + bs, gs, bes)
    return jax.nn.relu(o + sc)


def _cluster_max(h, width):
    return jnp.max(h.reshape(TSL, NODES_PER, width), axis=1)


def _body(x_ref, id_ref, mask_ref, *refs):
    out_ref = refs[-1]
    w = [r[...] for r in refs[:-1]]
    h = x_ref[...]
    i = 0
    for _ in range(NUM_SUB_LAYERS):
        h = _mlp_block(h, *w[i:i + 12])
        i += 12
        agg = _cluster_max(h, HID)
        aggb = jnp.broadcast_to(agg[:, None, :], (TSL, NODES_PER, HID))
        h = jnp.concatenate([h, aggb.reshape(NPG, HID)], axis=-1)
    Wl, bl = w[i], w[i + 1]
    i += 2
    h = jnp.dot(h, Wl, preferred_element_type=jnp.float32) + bl
    sub = _cluster_max(h, SUB_W)
    nrm = jnp.sqrt(jnp.sum(sub * sub, axis=-1, keepdims=True))
    sub = sub / jnp.maximum(nrm, 1e-12)
    ident = id_ref[...]
    Wqs, Wqi, bq, Wks, Wki, bk, Wvs, Wvi, bv = w[i:i + 9]
    q = (jnp.dot(sub, Wqs, preferred_element_type=jnp.float32)
         + jnp.dot(ident, Wqi, preferred_element_type=jnp.float32) + bq)
    k = (jnp.dot(sub, Wks, preferred_element_type=jnp.float32)
         + jnp.dot(ident, Wki, preferred_element_type=jnp.float32) + bk)
    v = (jnp.dot(sub, Wvs, preferred_element_type=jnp.float32)
         + jnp.dot(ident, Wvi, preferred_element_type=jnp.float32) + bv)
    scores = jax.lax.dot_general(q, k, (((1,), (1,)), ((), ())),
                                 preferred_element_type=jnp.float32)
    m = mask_ref[0]
    scores = jnp.where(m > 0, scores, -1e6)
    mx = jnp.max(scores, axis=-1, keepdims=True)
    e = jnp.exp(scores - mx)
    attn = e / jnp.sum(e, axis=-1, keepdims=True)
    out_ref[...] = jnp.dot(attn, v, preferred_element_type=jnp.float32)[None]


def kernel(x, cluster, edge_index, identifier, valid_len, params):
    del cluster, edge_index
    r = lambda a: a.reshape(1, -1)
    weights = []
    for p in params["sub_layers"]:
        weights += [p["W1"], r(p["b1"]), r(p["g1"]), r(p["be1"]),
                    p["W2"], r(p["b2"]), r(p["g2"]), r(p["be2"]),
                    p["Ws"], r(p["bs"]), r(p["gs"]), r(p["bes"])]
    weights += [params["Wl"], r(params["bl"])]
    for nm in ("q", "k", "v"):
        W = params["W" + nm]
        weights += [W[:SUB_W], W[SUB_W:], r(params["b" + nm])]
    mask = (jnp.arange(TSL, dtype=jnp.int32)[None, :]
            < valid_len[:, None]).astype(jnp.float32).reshape(BATCH, 1, TSL)
    in_specs = [
        pl.BlockSpec((NPG, IN_CH), lambda b: (b, 0)),
        pl.BlockSpec((TSL, 2), lambda b: (b, 0)),
        pl.BlockSpec((1, 1, TSL), lambda b: (b, 0, 0)),
    ] + [pl.BlockSpec(wt.shape, lambda b: (0, 0)) for wt in weights]
    return pl.pallas_call(
        _body,
        grid=(BATCH,),
        in_specs=in_specs,
        out_specs=pl.BlockSpec((1, TSL, GG_W), lambda b: (b, 0, 0)),
        out_shape=jax.ShapeDtypeStruct((BATCH, TSL, GG_W), jnp.float32),
        compiler_params=pltpu.CompilerParams(
            dimension_semantics=("arbitrary",)),
    )(x, identifier, mask, *weights)
